# Initial kernel scaffold; baseline (speedup 1.0000x reference)
import jax
import jax.numpy as jnp
from jax.experimental import pallas as pl


def kernel(x, x0, edge_index, W_l, b_l, W_r):
    raise NotImplementedError("write your pallas kernel here")


# speedup vs baseline: 4.4468x; 4.4468x over previous
import jax
import jax.numpy as jnp
from jax import lax
from jax.experimental import pallas as pl
from jax.experimental.pallas import tpu as pltpu
from jax.experimental.pallas import tpu_sc as plsc

N_NODES = 10000
D = 128
DH = D // 2

NC = 2
NS = 16

CH = 128
NCHUNK = 160
EPAD = NS * NCHUNK * CH
A_ROWS = 10240
ZROWS = A_ROWS // NS
CW = 8


def _sc_aggregate():
    mesh = plsc.VectorSubcoreMesh(core_axis_name="c", subcore_axis_name="s")
    out_type = (
        jax.ShapeDtypeStruct((NC, A_ROWS, DH), jnp.float32),
        jax.ShapeDtypeStruct((NC, A_ROWS, CW), jnp.float32),
    )
    scratch = [
        pltpu.VMEM((NCHUNK, CH), jnp.int32),
        pltpu.VMEM((NCHUNK, CH), jnp.int32),
        pltpu.VMEM((CH, DH), jnp.float32),
        pltpu.VMEM((CH, CW), jnp.float32),
        pltpu.VMEM_SHARED((A_ROWS, DH), jnp.float32),
        pltpu.VMEM_SHARED((A_ROWS, CW), jnp.float32),
    ]

    def body(xl_hbm, xr_hbm, src_hbm, dst_hbm, zf_hbm, zc_hbm, ones_hbm,
             outf_hbm, outc_hbm, src_v, dst_v, rows_v, ones_v, acc_s, cnt_s):
        c = lax.axis_index("c")
        s = lax.axis_index("s")

        pltpu.sync_copy(src_hbm.at[s], src_v)
        pltpu.sync_copy(dst_hbm.at[s], dst_v)
        pltpu.sync_copy(ones_hbm, ones_v)

        pltpu.sync_copy(zf_hbm, acc_s.at[pl.ds(s * ZROWS, ZROWS)])
        pltpu.sync_copy(zc_hbm, cnt_s.at[pl.ds(s * ZROWS, ZROWS)])
        plsc.subcore_barrier()

        def chunk(j, carry):
            @pl.when(c == 0)
            def _():
                pltpu.sync_copy(xl_hbm.at[src_v.at[j]], rows_v)

            @pl.when(c == 1)
            def _():
                pltpu.sync_copy(xr_hbm.at[src_v.at[j]], rows_v)

            pltpu.sync_copy(rows_v, acc_s.at[dst_v.at[j]], add=True)

            @pl.when(lax.rem(j, 2) == c)
            def _():
                pltpu.sync_copy(ones_v, cnt_s.at[dst_v.at[j]], add=True)

            return carry

        lax.fori_loop(0, NCHUNK, chunk, 0)
        plsc.subcore_barrier()

        r0 = s * ZROWS
        pltpu.sync_copy(acc_s.at[pl.ds(r0, ZROWS)],
                        outf_hbm.at[c, pl.ds(r0, ZROWS)])
        pltpu.sync_copy(cnt_s.at[pl.ds(r0, ZROWS)],
                        outc_hbm.at[c, pl.ds(r0, ZROWS)])

    return pl.kernel(body, out_type=out_type, mesh=mesh,
                     scratch_types=scratch,
                     compiler_params=pltpu.CompilerParams(
                         use_tc_tiling_on_sc=False))


_sc_agg = _sc_aggregate()


def _tc_tail(pf_ref, pc_ref, x_ref, wlt_ref, wrt_ref, b_ref, o_ref):
    agg = jnp.concatenate([pf_ref[0], pf_ref[1]], axis=1)
    cnt = (pc_ref[0] + pc_ref[1])[:, 0:1]
    mean = agg / jnp.maximum(cnt, 1.0)
    h = (jnp.dot(mean, wlt_ref[...], precision="highest",
                 preferred_element_type=jnp.float32)
         + b_ref[...]
         + jnp.dot(x_ref[...], wrt_ref[...], precision="highest",
                   preferred_element_type=jnp.float32))
    sq = jnp.sum(h * h, axis=1, keepdims=True)
    o_ref[...] = h * lax.rsqrt(jnp.maximum(sq, 1e-24))


@jax.jit
def kernel(x, x0, edge_index, W_l, b_l, W_r):
    del x0
    src = edge_index[0].astype(jnp.int32)
    dst = edge_index[1].astype(jnp.int32)
    pad = EPAD - src.shape[0]
    src_r = jnp.concatenate([src, jnp.zeros((pad,), jnp.int32)]
                            ).reshape(NS, NCHUNK, CH)
    dst_r = jnp.concatenate([dst, jnp.full((pad,), N_NODES, jnp.int32)]
                            ).reshape(NS, NCHUNK, CH)
    xl = x[:, :DH]
    xr = x[:, DH:]
    zf = jnp.zeros((ZROWS, DH), jnp.float32)
    zc = jnp.zeros((ZROWS, CW), jnp.float32)
    ones = jnp.ones((CH, CW), jnp.float32)

    pf, pc = _sc_agg(xl, xr, src_r, dst_r, zf, zc, ones)

    BM = 1000
    grid = (N_NODES // BM,)
    out = pl.pallas_call(
        _tc_tail,
        grid=grid,
        in_specs=[
            pl.BlockSpec((NC, BM, DH), lambda i: (0, i, 0)),
            pl.BlockSpec((NC, BM, CW), lambda i: (0, i, 0)),
            pl.BlockSpec((BM, D), lambda i: (i, 0)),
            pl.BlockSpec((D, D), lambda i: (0, 0)),
            pl.BlockSpec((D, D), lambda i: (0, 0)),
            pl.BlockSpec((1, D), lambda i: (0, 0)),
        ],
        out_specs=pl.BlockSpec((BM, D), lambda i: (i, 0)),
        out_shape=jax.ShapeDtypeStruct((N_NODES, D), jnp.float32),
    )(pf, pc, x, W_l.T, W_r.T, b_l[None, :])
    return out


# speedup vs baseline: 4.9022x; 1.1024x over previous
import jax
import jax.numpy as jnp
from jax import lax
from jax.experimental import pallas as pl
from jax.experimental.pallas import tpu as pltpu
from jax.experimental.pallas import tpu_sc as plsc

N_NODES = 10000
D = 128
DH = D // 2

NC = 2
NS = 16

CH = 128
NCHUNK = 160
EPAD = NS * NCHUNK * CH
A_ROWS = 10240
ZROWS = A_ROWS // NS
CW = 8
NBUF = 4
LOOKAHEAD = 2


def _sc_aggregate():
    mesh = plsc.VectorSubcoreMesh(core_axis_name="c", subcore_axis_name="s")
    out_type = (
        jax.ShapeDtypeStruct((NC, A_ROWS, DH), jnp.float32),
        jax.ShapeDtypeStruct((NC, A_ROWS, CW), jnp.float32),
    )
    scratch = (
        [pltpu.VMEM((NCHUNK, CH), jnp.int32)] * 2
        + [pltpu.VMEM((CH, DH), jnp.float32)] * NBUF
        + [pltpu.VMEM((CH, CW), jnp.float32)]
        + [pltpu.VMEM_SHARED((A_ROWS, DH), jnp.float32),
           pltpu.VMEM_SHARED((A_ROWS, CW), jnp.float32)]
        + [pltpu.SemaphoreType.DMA] * NBUF
        + [pltpu.SemaphoreType.DMA] * NBUF
        + [pltpu.SemaphoreType.DMA]
    )

    def body(xs_hbm, src_hbm, dst_hbm, zf_hbm, zc_hbm, ones_hbm,
             outf_hbm, outc_hbm, src_v, dst_v, *rest):
        rows = rest[:NBUF]
        ones_v = rest[NBUF]
        acc_s, cnt_s = rest[NBUF + 1], rest[NBUF + 2]
        gsem = rest[NBUF + 3:2 * NBUF + 3]
        ssem = rest[2 * NBUF + 3:3 * NBUF + 3]
        csem = rest[3 * NBUF + 3]

        c = lax.axis_index("c")
        s = lax.axis_index("s")

        pltpu.sync_copy(src_hbm.at[s], src_v)
        pltpu.sync_copy(dst_hbm.at[s], dst_v)
        pltpu.sync_copy(ones_hbm, ones_v)

        @pl.when(c == 1)
        def _():
            def bump(t, carry):
                i = lax.shift_right_logical(t, 3)
                k = lax.bitwise_and(t, 7) * 16
                src_v[i, pl.ds(k, 16)] = src_v[i, pl.ds(k, 16)] + 1
                return carry

            lax.fori_loop(0, NCHUNK * CH // 16, bump, 0)

        pltpu.sync_copy(zf_hbm, acc_s.at[pl.ds(s * ZROWS, ZROWS)])
        pltpu.sync_copy(zc_hbm, cnt_s.at[pl.ds(s * ZROWS, ZROWS)])
        plsc.subcore_barrier()

        def gather(j, b):
            pltpu.async_copy(xs_hbm.at[src_v.at[j]], rows[b], gsem[b])

        def gather_wait(j, b):
            pltpu.make_async_copy(xs_hbm.at[src_v.at[j]], rows[b],
                                  gsem[b]).wait()

        def scatter(j, b):
            pltpu.async_copy(rows[b], acc_s.at[dst_v.at[j]], ssem[b],
                             add=True)

        def scatter_wait(j, b):
            pltpu.make_async_copy(rows[b], acc_s.at[dst_v.at[j]],
                                  ssem[b]).wait()

        for b in range(LOOKAHEAD):
            gather(b, b)

        def group(k, carry):
            j0 = k * NBUF
            for b in range(NBUF):
                j = j0 + b
                gather_wait(j, b)
                scatter(j, b)

                @pl.when(lax.rem(j, 2) == c)
                def _():
                    pltpu.async_copy(ones_v, cnt_s.at[dst_v.at[j]], csem,
                                     add=True)

                nb = (b + LOOKAHEAD) % NBUF

                @pl.when(j >= LOOKAHEAD)
                def _():
                    scatter_wait(j - LOOKAHEAD, nb)

                @pl.when(j + LOOKAHEAD < NCHUNK)
                def _():
                    gather(j + LOOKAHEAD, nb)

            return carry

        lax.fori_loop(0, NCHUNK // NBUF, group, 0)

        for t in range(LOOKAHEAD):
            j = NCHUNK - LOOKAHEAD + t
            scatter_wait(j, j % NBUF)

        def cdrain(j, carry):
            pltpu.make_async_copy(ones_v, cnt_s.at[dst_v.at[0]],
                                  csem).wait()
            return carry

        lax.fori_loop(0, NCHUNK // 2, cdrain, 0)
        plsc.subcore_barrier()

        r0 = s * ZROWS
        pltpu.sync_copy(acc_s.at[pl.ds(r0, ZROWS)],
                        outf_hbm.at[c, pl.ds(r0, ZROWS)])
        pltpu.sync_copy(cnt_s.at[pl.ds(r0, ZROWS)],
                        outc_hbm.at[c, pl.ds(r0, ZROWS)])

    return pl.kernel(body, out_type=out_type, mesh=mesh,
                     scratch_types=scratch,
                     compiler_params=pltpu.CompilerParams(
                         use_tc_tiling_on_sc=False))


_sc_agg = _sc_aggregate()


def _tc_tail(pf_ref, pc_ref, x_ref, wlt_ref, wrt_ref, b_ref, o_ref):
    agg = jnp.concatenate([pf_ref[0], pf_ref[1]], axis=1)
    cnt = (pc_ref[0] + pc_ref[1])[:, 0:1]
    mean = agg / jnp.maximum(cnt, 1.0)
    h = (jnp.dot(mean, wlt_ref[...], precision="highest",
                 preferred_element_type=jnp.float32)
         + b_ref[...]
         + jnp.dot(x_ref[...], wrt_ref[...], precision="highest",
                   preferred_element_type=jnp.float32))
    sq = jnp.sum(h * h, axis=1, keepdims=True)
    o_ref[...] = h * lax.rsqrt(jnp.maximum(sq, 1e-24))


@jax.jit
def kernel(x, x0, edge_index, W_l, b_l, W_r):
    del x0
    src = edge_index[0].astype(jnp.int32)
    dst = edge_index[1].astype(jnp.int32)
    pad = EPAD - src.shape[0]
    src2 = jnp.concatenate([src * 2, jnp.zeros((pad,), jnp.int32)]
                           ).reshape(NS, NCHUNK, CH)
    dst_r = jnp.concatenate([dst, jnp.full((pad,), N_NODES, jnp.int32)]
                            ).reshape(NS, NCHUNK, CH)
    xs = x.reshape(2 * N_NODES, DH)
    zf = jnp.zeros((ZROWS, DH), jnp.float32)
    zc = jnp.zeros((ZROWS, CW), jnp.float32)
    ones = jnp.ones((CH, CW), jnp.float32)

    pf, pc = _sc_agg(xs, src2, dst_r, zf, zc, ones)

    BM = 1000
    grid = (N_NODES // BM,)
    out = pl.pallas_call(
        _tc_tail,
        grid=grid,
        in_specs=[
            pl.BlockSpec((NC, BM, DH), lambda i: (0, i, 0)),
            pl.BlockSpec((NC, BM, CW), lambda i: (0, i, 0)),
            pl.BlockSpec((BM, D), lambda i: (i, 0)),
            pl.BlockSpec((D, D), lambda i: (0, 0)),
            pl.BlockSpec((D, D), lambda i: (0, 0)),
            pl.BlockSpec((1, D), lambda i: (0, 0)),
        ],
        out_specs=pl.BlockSpec((BM, D), lambda i: (i, 0)),
        out_shape=jax.ShapeDtypeStruct((N_NODES, D), jnp.float32),
    )(pf, pc, x, W_l.T, W_r.T, b_l[None, :])
    return out
